# Initial kernel scaffold; baseline (speedup 1.0000x reference)
import jax
import jax.numpy as jnp
from jax.experimental import pallas as pl


def kernel(x, edge_index, edge_attr, nroi, batch_idx, W0_0, W1_0, b_0, g_0, be_0, W0_1, W1_1, b_1, g_1, be_1, W0_2, W1_2, b_2, g_2, be_2, Wm1, bm1, gm1, bem1, Wm2, bm2, gm2, bem2, Wm3, bm3):
    raise NotImplementedError("write your pallas kernel here")


# speedup vs baseline: 1.0819x; 1.0819x over previous
import jax
import jax.numpy as jnp
from jax.experimental import pallas as pl
from jax.experimental.pallas import tpu as pltpu

N = 10000
E = 320000
G = 100
EPS = 1e-5


def _lrelu(x):
    return jnp.where(x >= 0, x, 0.01 * x)


def _bn(x, g, b):
    m = jnp.mean(x, axis=0)
    v = jnp.var(x, axis=0)
    return (x - m) * jax.lax.rsqrt(v + EPS) * g + b


def _mlp_head_kernel(xx_ref, wm1_ref, bm1_ref, gm1_ref, bem1_ref,
                     wm2_ref, bm2_ref, gm2_ref, bem2_ref,
                     wm3_ref, bm3_ref, out_ref):
    xx = xx_ref[...]

    def bn_lrelu(z, g, b):
        m = jnp.mean(z, axis=0, keepdims=True)
        v = jnp.mean((z - m) * (z - m), axis=0, keepdims=True)
        zn = (z - m) * jax.lax.rsqrt(v + EPS) * g + b
        return jnp.where(zn >= 0, zn, 0.01 * zn)

    def mm(a, b):
        return jnp.dot(a.astype(jnp.bfloat16), b.astype(jnp.bfloat16),
                       preferred_element_type=jnp.float32)

    h = bn_lrelu(mm(xx, wm1_ref[...]) + bm1_ref[...], gm1_ref[...], bem1_ref[...])
    h = bn_lrelu(mm(h, wm2_ref[...]) + bm2_ref[...], gm2_ref[...], bem2_ref[...])
    out_ref[...] = mm(h, wm3_ref[...]) + bm3_ref[...]


def kernel(x, edge_index, edge_attr, nroi, batch_idx,
           W0_0, W1_0, b_0, g_0, be_0,
           W0_1, W1_1, b_1, g_1, be_1,
           W0_2, W1_2, b_2, g_2, be_2,
           Wm1, bm1, gm1, bem1,
           Wm2, bm2, gm2, bem2,
           Wm3, bm3):
    src = edge_index[0]
    dst = edge_index[1]

    deg = jax.ops.segment_sum(edge_attr, dst, num_segments=N)
    dis = jax.lax.rsqrt(jnp.maximum(deg, 1e-12))
    dis = jnp.where(deg > 0, dis, 0.0)

    mm = lambda a, b: jnp.dot(a.astype(jnp.bfloat16), b.astype(jnp.bfloat16),
                              preferred_element_type=jnp.float32)
    h = x
    for (W0, W1, b, g, be) in ((W0_0, W1_0, b_0, g_0, be_0),
                               (W0_1, W1_1, b_1, g_1, be_1),
                               (W0_2, W1_2, b_2, g_2, be_2)):
        norm = -dis[src] * edge_attr * dis[dst]
        tx1 = jax.ops.segment_sum(norm[:, None] * h[src], dst, num_segments=N)
        z = mm(h, W0) + mm(tx1, W1) + b
        h = _lrelu(_bn(z, g, be))

    x1 = jax.ops.segment_max(h, batch_idx, num_segments=G)
    x1 = jnp.where(jnp.isfinite(x1), x1, 0.0)
    cnt = jax.ops.segment_sum(jnp.ones((N,), jnp.float32), batch_idx, num_segments=G)
    x2 = jax.ops.segment_sum(h, batch_idx, num_segments=G) / jnp.maximum(cnt, 1.0)[:, None]
    xx = jnp.concatenate([x1, x2], axis=1)

    out = pl.pallas_call(
        _mlp_head_kernel,
        out_shape=jax.ShapeDtypeStruct((G, 8), jnp.float32),
    )(xx, Wm1, bm1, gm1, bem1, Wm2, bm2, gm2, bem2, Wm3, bm3)
    return out


# speedup vs baseline: 8.9712x; 8.2918x over previous
import functools

import jax
import jax.numpy as jnp
from jax import lax
from jax.experimental import pallas as pl
from jax.experimental.pallas import tpu as pltpu
from jax.experimental.pallas import tpu_sc as plsc

N = 10000
E = 320000
G = 100
EPS = 1e-5

NC = 2
NS = 16
NW = NC * NS
EW = E // NW
CH = 80
NCHUNK = EW // CH
D = 128

STRIPE = 640
LAST_STRIPE = N - 15 * STRIPE

NPAD = 10240
SLAB = NPAD // NS

_SC_MESH = plsc.VectorSubcoreMesh(core_axis_name="c", subcore_axis_name="s")
_SC_PARAMS = pltpu.CompilerParams(use_tc_tiling_on_sc=False)


def _mm(a, b):
    return jnp.dot(a.astype(jnp.bfloat16), b.astype(jnp.bfloat16),
                   preferred_element_type=jnp.float32)


def _bn_lrelu(z, g, be):
    m = jnp.mean(z, axis=0, keepdims=True)
    v = jnp.mean((z - m) * (z - m), axis=0, keepdims=True)
    zn = (z - m) * jax.lax.rsqrt(v + EPS) * g + be
    return jnp.where(zn >= 0, zn, 0.01 * zn)


def _sc_deg_body(dst_hbm, ew_hbm, out_hbm, ew_v, idxbuf, zbuf, acc_sh):
    c = lax.axis_index("c")
    s = lax.axis_index("s")
    wid = c * NS + s

    zeros16 = jnp.zeros((16,), jnp.float32)

    def _z(i, _):
        zbuf[pl.ds(i * 16, 16)] = zeros16
        return 0

    lax.fori_loop(0, STRIPE // 16, _z, 0)

    @pl.when(s < 15)
    def _():
        pltpu.sync_copy(zbuf, acc_sh.at[pl.ds(s * STRIPE, STRIPE)])

    @pl.when(s == 15)
    def _():
        pltpu.sync_copy(zbuf.at[pl.ds(0, LAST_STRIPE)],
                        acc_sh.at[pl.ds(15 * STRIPE, LAST_STRIPE)])

    plsc.subcore_barrier()

    pltpu.sync_copy(ew_hbm.at[wid], ew_v)

    def _chunk(k, _):
        pltpu.sync_copy(dst_hbm.at[wid, pl.ds(k * CH, CH)], idxbuf)
        pltpu.sync_copy(ew_v.at[pl.ds(k * CH, CH)], acc_sh.at[idxbuf],
                        add=True)
        return 0

    lax.fori_loop(0, NCHUNK, _chunk, 0)

    plsc.subcore_barrier()

    @pl.when(s < 15)
    def _():
        pltpu.sync_copy(acc_sh.at[pl.ds(s * STRIPE, STRIPE)],
                        out_hbm.at[c, pl.ds(s * STRIPE, STRIPE)])

    @pl.when(s == 15)
    def _():
        pltpu.sync_copy(acc_sh.at[pl.ds(15 * STRIPE, LAST_STRIPE)],
                        out_hbm.at[c, pl.ds(15 * STRIPE, LAST_STRIPE)])


_sc_deg = functools.partial(
    pl.kernel,
    out_type=jax.ShapeDtypeStruct((NC, N), jnp.float32),
    mesh=_SC_MESH,
    scratch_types=[
        pltpu.VMEM((EW,), jnp.float32),
        pltpu.VMEM((CH,), jnp.int32),
        pltpu.VMEM((STRIPE,), jnp.float32),
        pltpu.VMEM_SHARED((N,), jnp.float32),
    ],
    compiler_params=_SC_PARAMS,
)(_sc_deg_body)


def _sc_layer_body(hd_hbm, src_hbm, dst_hbm, ew_hbm, out_hbm,
                   src_v, ew_v, idxbuf, rows_v, zbuf, acc_sh, sem):
    c = lax.axis_index("c")
    s = lax.axis_index("s")
    wid = c * NS + s

    zeros16 = jnp.zeros((16,), jnp.float32)

    for i in range(CH):
        for kk in range(D // 16):
            zbuf[i, pl.ds(kk * 16, 16)] = zeros16

    nrep = STRIPE // CH
    last_nrep = LAST_STRIPE // CH

    def _zcopy(j, _):
        pltpu.sync_copy(zbuf, acc_sh.at[pl.ds(s * STRIPE + j * CH, CH), :])
        return 0

    @pl.when(s < 15)
    def _():
        lax.fori_loop(0, nrep, _zcopy, 0)

    @pl.when(s == 15)
    def _():
        lax.fori_loop(0, last_nrep, _zcopy, 0)

    pltpu.sync_copy(src_hbm.at[wid], src_v)
    pltpu.sync_copy(ew_hbm.at[wid], ew_v)

    plsc.subcore_barrier()

    def _chunk(k, _):
        pltpu.sync_copy(dst_hbm.at[wid, pl.ds(k * CH, CH)], idxbuf)
        pltpu.async_copy(hd_hbm.at[src_v.at[pl.ds(k * CH, CH)]], rows_v,
                         sem).wait()

        def _grp(i16, _):
            ev = ew_v[pl.ds(k * CH + i16 * 16, 16)]
            for j in range(16):
                sc = ev[j]
                base = i16 * 16 + j
                for kk in range(D // 16):
                    rows_v[base, pl.ds(kk * 16, 16)] = (
                        rows_v[base, pl.ds(kk * 16, 16)] * sc)
            return 0

        lax.fori_loop(0, CH // 16, _grp, 0)
        pltpu.sync_copy(rows_v, acc_sh.at[idxbuf], add=True)
        return 0

    lax.fori_loop(0, NCHUNK, _chunk, 0)

    plsc.subcore_barrier()

    def _dump(j, _):
        pltpu.sync_copy(acc_sh.at[pl.ds(s * STRIPE + j * CH, CH), :],
                        out_hbm.at[c, pl.ds(s * STRIPE + j * CH, CH), :])
        return 0

    @pl.when(s < 15)
    def _():
        lax.fori_loop(0, nrep, _dump, 0)

    @pl.when(s == 15)
    def _():
        lax.fori_loop(0, last_nrep, _dump, 0)


_sc_layer = functools.partial(
    pl.kernel,
    out_type=jax.ShapeDtypeStruct((NC, N, D), jnp.float32),
    mesh=_SC_MESH,
    scratch_types=[
        pltpu.VMEM((EW,), jnp.int32),
        pltpu.VMEM((EW,), jnp.float32),
        pltpu.VMEM((CH,), jnp.int32),
        pltpu.VMEM((CH, D), jnp.float32),
        pltpu.VMEM((CH, D), jnp.float32),
        pltpu.VMEM_SHARED((N, D), jnp.float32),
        pltpu.SemaphoreType.DMA,
    ],
    compiler_params=_SC_PARAMS,
)(_sc_layer_body)


def _sc_pool_body(h_hbm, bidx_hbm, outmax_hbm, outsum_hbm,
                  rows_v, bidx_v, accmax_v, accsum_v):
    c = lax.axis_index("c")
    s = lax.axis_index("s")

    neg = jnp.full((16,), -3.0e38, jnp.float32)
    zeros16 = jnp.zeros((16,), jnp.float32)

    def _init(g, _):
        accmax_v[g, :] = neg
        accsum_v[g, :] = zeros16
        return 0

    lax.fori_loop(0, G + 4, _init, 0)

    pltpu.sync_copy(h_hbm.at[pl.ds(s * SLAB, SLAB), pl.ds(c * 16, 16)],
                    rows_v)
    pltpu.sync_copy(bidx_hbm.at[pl.ds(s * SLAB, SLAB)], bidx_v)

    def _grpr(i16, _):
        bv = bidx_v[pl.ds(i16 * 16, 16)]
        for j in range(16):
            g = bv[j]
            i = i16 * 16 + j
            r = rows_v[i, :]
            accmax_v[g, :] = jnp.maximum(accmax_v[g, :], r)
            accsum_v[g, :] = accsum_v[g, :] + r
        return 0

    lax.fori_loop(0, SLAB // 16, _grpr, 0)

    pltpu.sync_copy(accmax_v.at[pl.ds(0, G), :],
                    outmax_hbm.at[s, :, pl.ds(c * 16, 16)])
    pltpu.sync_copy(accsum_v.at[pl.ds(0, G), :],
                    outsum_hbm.at[s, :, pl.ds(c * 16, 16)])


_sc_pool = functools.partial(
    pl.kernel,
    out_type=(jax.ShapeDtypeStruct((NS, G, 32), jnp.float32),
              jax.ShapeDtypeStruct((NS, G, 32), jnp.float32)),
    mesh=_SC_MESH,
    scratch_types=[
        pltpu.VMEM((SLAB, 16), jnp.float32),
        pltpu.VMEM((SLAB,), jnp.int32),
        pltpu.VMEM((G + 4, 16), jnp.float32),
        pltpu.VMEM((G + 4, 16), jnp.float32),
    ],
    compiler_params=_SC_PARAMS,
)(_sc_pool_body)


def _tc_dis_body(degp_ref, x_ref, dis_ref, hd_ref):
    deg = degp_ref[0] + degp_ref[1]
    dis = jax.lax.rsqrt(jnp.maximum(deg, 1e-12))
    dis = jnp.where(deg > 0, dis, 0.0)
    dis_ref[...] = dis
    hd_ref[...] = dis[:, None] * x_ref[...]


_tc_dis = pl.pallas_call(
    _tc_dis_body,
    out_shape=(jax.ShapeDtypeStruct((N,), jnp.float32),
               jax.ShapeDtypeStruct((N, 128), jnp.float32)),
)


def _tc_layer_body(din, dout, emit_hd, h_ref, p_ref, dis_ref, w0_ref, w1_ref,
                   b_ref, g_ref, be_ref, *out_refs):
    dis = dis_ref[...]
    h = h_ref[...][:, :din]
    tx = (-dis)[:, None] * (p_ref[0] + p_ref[1])[:, :din]
    z = _mm(h, w0_ref[...]) + _mm(tx, w1_ref[...]) + b_ref[...]
    hn = _bn_lrelu(z, g_ref[...], be_ref[...])

    out_ref = out_refs[0]
    rows, cols = out_ref.shape
    if cols > dout:
        out_ref[:, pl.ds(0, dout)] = hn
        out_ref[:, pl.ds(dout, cols - dout)] = jnp.zeros(
            (rows, cols - dout), jnp.float32)
        if emit_hd:
            hd_ref = out_refs[1]
            hd_ref[:, pl.ds(0, dout)] = dis[:, None] * hn
            hd_ref[:, pl.ds(dout, cols - dout)] = jnp.zeros(
                (rows, cols - dout), jnp.float32)
    elif rows > N:
        out_ref[pl.ds(0, N), :] = hn
        out_ref[pl.ds(N, rows - N), :] = jnp.zeros(
            (rows - N, cols), jnp.float32)
    else:
        out_ref[...] = hn
        if emit_hd:
            out_refs[1][...] = dis[:, None] * hn


def _make_tc_layer(din, dout, out_rows, out_cols, emit_hd):
    shapes = [jax.ShapeDtypeStruct((out_rows, out_cols), jnp.float32)]
    if emit_hd:
        shapes.append(jax.ShapeDtypeStruct((out_rows, out_cols), jnp.float32))
    return pl.pallas_call(
        functools.partial(_tc_layer_body, din, dout, emit_hd),
        out_shape=tuple(shapes) if emit_hd else shapes[0],
    )


_tc_layer0 = _make_tc_layer(128, 128, N, 128, True)
_tc_layer1 = _make_tc_layer(128, 64, N, 128, True)
_tc_layer2 = _make_tc_layer(64, 32, NPAD, 32, False)


def _tc_head_body(maxp_ref, sump_ref, bidx_ref,
                  wm1_ref, bm1_ref, gm1_ref, bem1_ref,
                  wm2_ref, bm2_ref, gm2_ref, bem2_ref,
                  wm3_ref, bm3_ref, out_ref):
    bidx = bidx_ref[...]
    gids = jax.lax.broadcasted_iota(jnp.int32, (G, N), 0)
    onehot = (gids == bidx[None, :]).astype(jnp.float32)
    cnt = jnp.sum(onehot, axis=1)

    x1 = jnp.max(maxp_ref[...], axis=0)
    x1 = jnp.where(cnt[:, None] > 0, x1, 0.0)
    x2 = jnp.sum(sump_ref[...], axis=0) / jnp.maximum(cnt, 1.0)[:, None]
    xx = jnp.concatenate([x1, x2], axis=1)

    h = _bn_lrelu(_mm(xx, wm1_ref[...]) + bm1_ref[...],
                  gm1_ref[...], bem1_ref[...])
    h = _bn_lrelu(_mm(h, wm2_ref[...]) + bm2_ref[...],
                  gm2_ref[...], bem2_ref[...])
    out_ref[...] = _mm(h, wm3_ref[...]) + bm3_ref[...]


_tc_head = pl.pallas_call(
    _tc_head_body,
    out_shape=jax.ShapeDtypeStruct((G, 8), jnp.float32),
)


def kernel(x, edge_index, edge_attr, nroi, batch_idx,
           W0_0, W1_0, b_0, g_0, be_0,
           W0_1, W1_1, b_1, g_1, be_1,
           W0_2, W1_2, b_2, g_2, be_2,
           Wm1, bm1, gm1, bem1,
           Wm2, bm2, gm2, bem2,
           Wm3, bm3):
    src_r = edge_index[0].astype(jnp.int32).reshape(NW, EW)
    dst_r = edge_index[1].astype(jnp.int32).reshape(NW, EW)
    ew_r = edge_attr.reshape(NW, EW)

    deg_parts = _sc_deg(dst_r, ew_r)
    dis, hd0 = _tc_dis(deg_parts, x)

    p0 = _sc_layer(hd0, src_r, dst_r, ew_r)
    h1, hd1 = _tc_layer0(x, p0, dis, W0_0, W1_0, b_0, g_0, be_0)

    p1 = _sc_layer(hd1, src_r, dst_r, ew_r)
    h2p, hd2p = _tc_layer1(h1, p1, dis, W0_1, W1_1, b_1, g_1, be_1)

    p2 = _sc_layer(hd2p, src_r, dst_r, ew_r)
    h3p = _tc_layer2(h2p, p2, dis, W0_2, W1_2, b_2, g_2, be_2)

    bidxp = jnp.pad(batch_idx.astype(jnp.int32), (0, NPAD - N),
                    constant_values=G)
    maxp, sump = _sc_pool(h3p, bidxp)

    out = _tc_head(maxp, sump, batch_idx.astype(jnp.int32),
                   Wm1, bm1, gm1, bem1, Wm2, bm2, gm2, bem2, Wm3, bm3)
    return out


# speedup vs baseline: 16.4472x; 1.8333x over previous
import functools

import jax
import jax.numpy as jnp
from jax import lax
from jax.experimental import pallas as pl
from jax.experimental.pallas import tpu as pltpu
from jax.experimental.pallas import tpu_sc as plsc

N = 10000
E = 320000
G = 100
EPS = 1e-5

NC = 2
NS = 16
NW = NC * NS
EW = E // NW
CH = 80
NCHUNK = EW // CH
D = 128

STRIPE = 640
LAST_STRIPE = N - 15 * STRIPE

NPAD = 10240
SLAB = NPAD // NS

_SC_MESH = plsc.VectorSubcoreMesh(core_axis_name="c", subcore_axis_name="s")
_SC_PARAMS = pltpu.CompilerParams(use_tc_tiling_on_sc=False)


def _mm(a, b):
    return jnp.dot(a.astype(jnp.bfloat16), b.astype(jnp.bfloat16),
                   preferred_element_type=jnp.float32)


def _bn_lrelu(z, g, be):
    m = jnp.mean(z, axis=0, keepdims=True)
    v = jnp.mean((z - m) * (z - m), axis=0, keepdims=True)
    zn = (z - m) * jax.lax.rsqrt(v + EPS) * g + be
    return jnp.where(zn >= 0, zn, 0.01 * zn)


def _sc_deg_body(dst_hbm, ew_hbm, out_hbm, ew_v, idxbuf, zbuf, acc_sh):
    c = lax.axis_index("c")
    s = lax.axis_index("s")
    wid = c * NS + s

    zeros16 = jnp.zeros((16,), jnp.float32)

    def _z(i, _):
        zbuf[pl.ds(i * 16, 16)] = zeros16
        return 0

    lax.fori_loop(0, STRIPE // 16, _z, 0)

    @pl.when(s < 15)
    def _():
        pltpu.sync_copy(zbuf, acc_sh.at[pl.ds(s * STRIPE, STRIPE)])

    @pl.when(s == 15)
    def _():
        pltpu.sync_copy(zbuf.at[pl.ds(0, LAST_STRIPE)],
                        acc_sh.at[pl.ds(15 * STRIPE, LAST_STRIPE)])

    plsc.subcore_barrier()

    pltpu.sync_copy(ew_hbm.at[wid], ew_v)

    def _chunk(k, _):
        pltpu.sync_copy(dst_hbm.at[wid, pl.ds(k * CH, CH)], idxbuf)
        pltpu.sync_copy(ew_v.at[pl.ds(k * CH, CH)], acc_sh.at[idxbuf],
                        add=True)
        return 0

    lax.fori_loop(0, NCHUNK, _chunk, 0)

    plsc.subcore_barrier()

    @pl.when(s < 15)
    def _():
        pltpu.sync_copy(acc_sh.at[pl.ds(s * STRIPE, STRIPE)],
                        out_hbm.at[c, pl.ds(s * STRIPE, STRIPE)])

    @pl.when(s == 15)
    def _():
        pltpu.sync_copy(acc_sh.at[pl.ds(15 * STRIPE, LAST_STRIPE)],
                        out_hbm.at[c, pl.ds(15 * STRIPE, LAST_STRIPE)])


_sc_deg = functools.partial(
    pl.kernel,
    out_type=jax.ShapeDtypeStruct((NC, N), jnp.float32),
    mesh=_SC_MESH,
    scratch_types=[
        pltpu.VMEM((EW,), jnp.float32),
        pltpu.VMEM((CH,), jnp.int32),
        pltpu.VMEM((STRIPE,), jnp.float32),
        pltpu.VMEM_SHARED((N,), jnp.float32),
    ],
    compiler_params=_SC_PARAMS,
)(_sc_deg_body)


def _sc_layer_body(hd_hbm, src_hbm, dst_hbm, ew_hbm, out_hbm,
                   src_v, ew_v, idx0, idx1, rows0, rows1, zbuf, acc_sh,
                   isem0, isem1, gsem0, gsem1, ssem0, ssem1):
    c = lax.axis_index("c")
    s = lax.axis_index("s")
    wid = c * NS + s

    zeros16 = jnp.zeros((16,), jnp.float32)

    for i in range(CH):
        for kk in range(D // 16):
            zbuf[i, pl.ds(kk * 16, 16)] = zeros16

    nrep = STRIPE // CH
    last_nrep = LAST_STRIPE // CH

    def _zcopy(j, _):
        pltpu.sync_copy(zbuf, acc_sh.at[pl.ds(s * STRIPE + j * CH, CH), :])
        return 0

    @pl.when(s < 15)
    def _():
        lax.fori_loop(0, nrep, _zcopy, 0)

    @pl.when(s == 15)
    def _():
        lax.fori_loop(0, last_nrep, _zcopy, 0)

    pltpu.sync_copy(src_hbm.at[wid], src_v)
    pltpu.sync_copy(ew_hbm.at[wid], ew_v)

    plsc.subcore_barrier()

    def _issue(cc, idx_b, rows_b, isem_b, gsem_b, ssem_b):
        @pl.when(cc >= 2)
        def _():
            pltpu.make_async_copy(rows_b, acc_sh.at[idx_b], ssem_b).wait()

        pltpu.async_copy(dst_hbm.at[wid, pl.ds(cc * CH, CH)], idx_b, isem_b)
        pltpu.async_copy(hd_hbm.at[src_v.at[pl.ds(cc * CH, CH)]], rows_b,
                         gsem_b)

    def _consume(cc, idx_b, rows_b, isem_b, gsem_b, ssem_b):
        pltpu.make_async_copy(dst_hbm.at[wid, pl.ds(cc * CH, CH)], idx_b,
                              isem_b).wait()
        pltpu.make_async_copy(hd_hbm.at[src_v.at[pl.ds(cc * CH, CH)]],
                              rows_b, gsem_b).wait()

        def _grp(i16, _):
            ev = ew_v[pl.ds(cc * CH + i16 * 16, 16)]
            for j in range(16):
                sc = ev[j]
                base = i16 * 16 + j
                for kk in range(D // 16):
                    rows_b[base, pl.ds(kk * 16, 16)] = (
                        rows_b[base, pl.ds(kk * 16, 16)] * sc)
            return 0

        lax.fori_loop(0, CH // 16, _grp, 0)
        pltpu.async_copy(rows_b, acc_sh.at[idx_b], ssem_b, add=True)

    b0 = (idx0, rows0, isem0, gsem0, ssem0)
    b1 = (idx1, rows1, isem1, gsem1, ssem1)

    _issue(0, *b0)

    def _pair(kk, _):
        c0 = kk * 2

        @pl.when(c0 + 1 < NCHUNK)
        def _():
            _issue(c0 + 1, *b1)

        _consume(c0, *b0)

        @pl.when(c0 + 2 < NCHUNK)
        def _():
            _issue(c0 + 2, *b0)

        @pl.when(c0 + 1 < NCHUNK)
        def _():
            _consume(c0 + 1, *b1)

        return 0

    lax.fori_loop(0, (NCHUNK + 1) // 2, _pair, 0)

    pltpu.make_async_copy(rows0, acc_sh.at[idx0], ssem0).wait()
    pltpu.make_async_copy(rows1, acc_sh.at[idx1], ssem1).wait()

    plsc.subcore_barrier()

    def _dump(j, _):
        pltpu.sync_copy(acc_sh.at[pl.ds(s * STRIPE + j * CH, CH), :],
                        out_hbm.at[c, pl.ds(s * STRIPE + j * CH, CH), :])
        return 0

    @pl.when(s < 15)
    def _():
        lax.fori_loop(0, nrep, _dump, 0)

    @pl.when(s == 15)
    def _():
        lax.fori_loop(0, last_nrep, _dump, 0)


_sc_layer = functools.partial(
    pl.kernel,
    out_type=jax.ShapeDtypeStruct((NC, N, D), jnp.float32),
    mesh=_SC_MESH,
    scratch_types=[
        pltpu.VMEM((EW,), jnp.int32),
        pltpu.VMEM((EW,), jnp.float32),
        pltpu.VMEM((CH,), jnp.int32),
        pltpu.VMEM((CH,), jnp.int32),
        pltpu.VMEM((CH, D), jnp.float32),
        pltpu.VMEM((CH, D), jnp.float32),
        pltpu.VMEM((CH, D), jnp.float32),
        pltpu.VMEM_SHARED((N, D), jnp.float32),
        pltpu.SemaphoreType.DMA,
        pltpu.SemaphoreType.DMA,
        pltpu.SemaphoreType.DMA,
        pltpu.SemaphoreType.DMA,
        pltpu.SemaphoreType.DMA,
        pltpu.SemaphoreType.DMA,
    ],
    compiler_params=_SC_PARAMS,
)(_sc_layer_body)


def _sc_pool_body(h_hbm, bidx_hbm, outmax_hbm, outsum_hbm,
                  rows_v, bidx_v, accmax_v, accsum_v):
    c = lax.axis_index("c")
    s = lax.axis_index("s")

    neg = jnp.full((16,), -3.0e38, jnp.float32)
    zeros16 = jnp.zeros((16,), jnp.float32)

    def _init(g, _):
        accmax_v[g, :] = neg
        accsum_v[g, :] = zeros16
        return 0

    lax.fori_loop(0, G + 4, _init, 0)

    pltpu.sync_copy(h_hbm.at[pl.ds(s * SLAB, SLAB), pl.ds(c * 16, 16)],
                    rows_v)
    pltpu.sync_copy(bidx_hbm.at[pl.ds(s * SLAB, SLAB)], bidx_v)

    def _grpr(i16, _):
        bv = bidx_v[pl.ds(i16 * 16, 16)]
        for j in range(16):
            g = bv[j]
            i = i16 * 16 + j
            r = rows_v[i, :]
            accmax_v[g, :] = jnp.maximum(accmax_v[g, :], r)
            accsum_v[g, :] = accsum_v[g, :] + r
        return 0

    lax.fori_loop(0, SLAB // 16, _grpr, 0)

    pltpu.sync_copy(accmax_v.at[pl.ds(0, G), :],
                    outmax_hbm.at[s, :, pl.ds(c * 16, 16)])
    pltpu.sync_copy(accsum_v.at[pl.ds(0, G), :],
                    outsum_hbm.at[s, :, pl.ds(c * 16, 16)])


_sc_pool = functools.partial(
    pl.kernel,
    out_type=(jax.ShapeDtypeStruct((NS, G, 32), jnp.float32),
              jax.ShapeDtypeStruct((NS, G, 32), jnp.float32)),
    mesh=_SC_MESH,
    scratch_types=[
        pltpu.VMEM((SLAB, 16), jnp.float32),
        pltpu.VMEM((SLAB,), jnp.int32),
        pltpu.VMEM((G + 4, 16), jnp.float32),
        pltpu.VMEM((G + 4, 16), jnp.float32),
    ],
    compiler_params=_SC_PARAMS,
)(_sc_pool_body)


def _tc_dis_body(degp_ref, x_ref, dis_ref, hd_ref):
    deg = degp_ref[0] + degp_ref[1]
    dis = jax.lax.rsqrt(jnp.maximum(deg, 1e-12))
    dis = jnp.where(deg > 0, dis, 0.0)
    dis_ref[...] = dis
    hd_ref[...] = dis[:, None] * x_ref[...]


_tc_dis = pl.pallas_call(
    _tc_dis_body,
    out_shape=(jax.ShapeDtypeStruct((N,), jnp.float32),
               jax.ShapeDtypeStruct((N, 128), jnp.float32)),
)


def _tc_layer_body(din, dout, emit_hd, h_ref, p_ref, dis_ref, w0_ref, w1_ref,
                   b_ref, g_ref, be_ref, *out_refs):
    dis = dis_ref[...]
    h = h_ref[...][:, :din]
    tx = (-dis)[:, None] * (p_ref[0] + p_ref[1])[:, :din]
    z = _mm(h, w0_ref[...]) + _mm(tx, w1_ref[...]) + b_ref[...]
    hn = _bn_lrelu(z, g_ref[...], be_ref[...])

    out_ref = out_refs[0]
    rows, cols = out_ref.shape
    if cols > dout:
        out_ref[:, pl.ds(0, dout)] = hn
        out_ref[:, pl.ds(dout, cols - dout)] = jnp.zeros(
            (rows, cols - dout), jnp.float32)
        if emit_hd:
            hd_ref = out_refs[1]
            hd_ref[:, pl.ds(0, dout)] = dis[:, None] * hn
            hd_ref[:, pl.ds(dout, cols - dout)] = jnp.zeros(
                (rows, cols - dout), jnp.float32)
    elif rows > N:
        out_ref[pl.ds(0, N), :] = hn
        out_ref[pl.ds(N, rows - N), :] = jnp.zeros(
            (rows - N, cols), jnp.float32)
    else:
        out_ref[...] = hn
        if emit_hd:
            out_refs[1][...] = dis[:, None] * hn


def _make_tc_layer(din, dout, out_rows, out_cols, emit_hd):
    shapes = [jax.ShapeDtypeStruct((out_rows, out_cols), jnp.float32)]
    if emit_hd:
        shapes.append(jax.ShapeDtypeStruct((out_rows, out_cols), jnp.float32))
    return pl.pallas_call(
        functools.partial(_tc_layer_body, din, dout, emit_hd),
        out_shape=tuple(shapes) if emit_hd else shapes[0],
    )


_tc_layer0 = _make_tc_layer(128, 128, N, 128, True)
_tc_layer1 = _make_tc_layer(128, 64, N, 128, True)
_tc_layer2 = _make_tc_layer(64, 32, NPAD, 32, False)


def _tc_head_body(maxp_ref, sump_ref, bidx_ref,
                  wm1_ref, bm1_ref, gm1_ref, bem1_ref,
                  wm2_ref, bm2_ref, gm2_ref, bem2_ref,
                  wm3_ref, bm3_ref, out_ref):
    bidx = bidx_ref[...]
    gids = jax.lax.broadcasted_iota(jnp.int32, (G, N), 0)
    onehot = (gids == bidx[None, :]).astype(jnp.float32)
    cnt = jnp.sum(onehot, axis=1)

    x1 = jnp.max(maxp_ref[...], axis=0)
    x1 = jnp.where(cnt[:, None] > 0, x1, 0.0)
    x2 = jnp.sum(sump_ref[...], axis=0) / jnp.maximum(cnt, 1.0)[:, None]
    xx = jnp.concatenate([x1, x2], axis=1)

    h = _bn_lrelu(_mm(xx, wm1_ref[...]) + bm1_ref[...],
                  gm1_ref[...], bem1_ref[...])
    h = _bn_lrelu(_mm(h, wm2_ref[...]) + bm2_ref[...],
                  gm2_ref[...], bem2_ref[...])
    out_ref[...] = _mm(h, wm3_ref[...]) + bm3_ref[...]


_tc_head = pl.pallas_call(
    _tc_head_body,
    out_shape=jax.ShapeDtypeStruct((G, 8), jnp.float32),
)


def kernel(x, edge_index, edge_attr, nroi, batch_idx,
           W0_0, W1_0, b_0, g_0, be_0,
           W0_1, W1_1, b_1, g_1, be_1,
           W0_2, W1_2, b_2, g_2, be_2,
           Wm1, bm1, gm1, bem1,
           Wm2, bm2, gm2, bem2,
           Wm3, bm3):
    src_r = edge_index[0].astype(jnp.int32).reshape(NW, EW)
    dst_r = edge_index[1].astype(jnp.int32).reshape(NW, EW)
    ew_r = edge_attr.reshape(NW, EW)

    deg_parts = _sc_deg(dst_r, ew_r)
    dis, hd0 = _tc_dis(deg_parts, x)

    p0 = _sc_layer(hd0, src_r, dst_r, ew_r)
    h1, hd1 = _tc_layer0(x, p0, dis, W0_0, W1_0, b_0, g_0, be_0)

    p1 = _sc_layer(hd1, src_r, dst_r, ew_r)
    h2p, hd2p = _tc_layer1(h1, p1, dis, W0_1, W1_1, b_1, g_1, be_1)

    p2 = _sc_layer(hd2p, src_r, dst_r, ew_r)
    h3p = _tc_layer2(h2p, p2, dis, W0_2, W1_2, b_2, g_2, be_2)

    bidxp = jnp.pad(batch_idx.astype(jnp.int32), (0, NPAD - N),
                    constant_values=G)
    maxp, sump = _sc_pool(h3p, bidxp)

    out = _tc_head(maxp, sump, batch_idx.astype(jnp.int32),
                   Wm1, bm1, gm1, bem1, Wm2, bm2, gm2, bem2, Wm3, bm3)
    return out
